# Initial kernel scaffold; baseline (speedup 1.0000x reference)
import jax
import jax.numpy as jnp
from jax.experimental import pallas as pl


def kernel(x, active_indices):
    raise NotImplementedError("write your pallas kernel here")


# speedup vs baseline: 1.4272x; 1.4272x over previous
import functools

import jax
import jax.numpy as jnp
from jax import lax
from jax.experimental import pallas as pl
from jax.experimental.pallas import tpu as pltpu, tpu_sc as plsc

N = 4096
BH = BW = 32
GRID = N // BH
NNZ = 1638
NW = 32
PER_W = 52
PER_W_PAD = 64
LAST_W_BLOCKS = NNZ - (NW - 1) * PER_W
ROWS_PER_W = PER_W * BH
CHUNK = 128
N_CHUNKS = ROWS_PER_W // CHUNK

_mesh = plsc.VectorSubcoreMesh(core_axis_name="c", subcore_axis_name="s")


@functools.partial(
    pl.kernel,
    out_type=jax.ShapeDtypeStruct((NNZ * BH, BW), jnp.float32),
    mesh=_mesh,
    compiler_params=pltpu.CompilerParams(use_tc_tiling_on_sc=False),
    scratch_types=[
        pltpu.VMEM((PER_W_PAD,), jnp.int32),
        pltpu.VMEM((PER_W_PAD,), jnp.int32),
        pltpu.VMEM((PER_W_PAD * BH,), jnp.int32),
        pltpu.VMEM((ROWS_PER_W, BW), jnp.float32),
        pltpu.SemaphoreType.DMA,
    ],
)
def _gather_blocks(tbl, r2d, c2d, out, rows_v, cols_v, idx_v, buf, sem):
    wid = lax.axis_index("s") * 2 + lax.axis_index("c")
    pltpu.sync_copy(r2d.at[wid], rows_v)
    pltpu.sync_copy(c2d.at[wid], cols_v)

    lo = lax.iota(jnp.int32, 16) * GRID
    hi = lo + 16 * GRID
    for j in range(PER_W_PAD // 16):
        r16 = rows_v[pl.ds(j * 16, 16)]
        c16 = cols_v[pl.ds(j * 16, 16)]
        base16 = r16 * (BH * GRID) + c16
        for k in range(16):
            t = j * 16 + k
            if t >= PER_W:
                break
            b = base16[k]
            idx_v[pl.ds(t * BH, 16)] = b + lo
            idx_v[pl.ds(t * BH + 16, 16)] = b + hi

    copies = [
        pltpu.async_copy(tbl.at[idx_v.at[pl.ds(j * CHUNK, CHUNK)]],
                         buf.at[pl.ds(j * CHUNK, CHUNK)], sem)
        for j in range(N_CHUNKS)
    ]
    for cp in copies:
        cp.wait()

    @pl.when(wid < NW - 1)
    def _():
        pltpu.sync_copy(buf, out.at[pl.ds(wid * ROWS_PER_W, ROWS_PER_W)])

    @pl.when(wid == NW - 1)
    def _():
        last = LAST_W_BLOCKS * BH
        pltpu.sync_copy(buf.at[pl.ds(0, last)],
                        out.at[pl.ds((NW - 1) * ROWS_PER_W, last)])


def kernel(x, active_indices):
    ai = active_indices.astype(jnp.int32)
    pad_len = (NW - 1) * PER_W + PER_W_PAD
    rpad = jnp.zeros((pad_len,), jnp.int32).at[:NNZ].set(ai[:, 0])
    cpad = jnp.zeros((pad_len,), jnp.int32).at[:NNZ].set(ai[:, 1])
    win = (jnp.arange(NW, dtype=jnp.int32)[:, None] * PER_W
           + jnp.arange(PER_W_PAD, dtype=jnp.int32)[None, :])
    tbl = x.reshape(N * GRID, BW)
    out = _gather_blocks(tbl, rpad[win], cpad[win])
    return out.reshape(NNZ, BH, BW)
